# baseline (speedup 1.0000x reference)
import numpy as np
import jax
import jax.numpy as jnp
from jax import lax
from jax.experimental import pallas as pl
from jax.experimental.pallas import tpu as pltpu

B, T, D = 32, 2048, 768
TOP_M = 8
NEG = -1e30
K_LOW = 1432
_POS = np.float32(0.7) * np.float32(2047.0)
HW = float(_POS - np.float32(np.floor(_POS)))
LW = float(np.float32(1.0) - np.float32(HW))


def _shift_r(x, k):
    r = x.shape[0]
    return jnp.concatenate([jnp.zeros((r, k), x.dtype), x[:, :T - k]], axis=1)


def _shift_l(x, k):
    r = x.shape[0]
    return jnp.concatenate([x[:, k:], jnp.zeros((r, k), x.dtype)], axis=1)


def _maxscan_hs(x):
    y = x
    k = 1
    while k < T:
        r = x.shape[0]
        sh = jnp.concatenate([jnp.full((r, k), -1.0, x.dtype), y[:, :T - k]], axis=1)
        y = jnp.maximum(y, sh)
        k *= 2
    return y


def _segsum_hs(x, rid):
    y = x
    k = 1
    while k < T:
        same = jnp.where(_shift_r(rid, k) == rid, jnp.float32(1.0), jnp.float32(0.0))
        y = y + same * _shift_r(y, k)
        k *= 2
    return y


def _weights_allbatch(w):
    one = jnp.float32(1.0)
    zero = jnp.float32(0.0)
    wr = w.astype(jnp.bfloat16).astype(jnp.float32)
    c = jnp.float32(0.2001953125)

    s_m2 = jnp.concatenate([wr[:, :1], wr[:, :1], wr[:, :T - 2]], axis=1)
    s_m1 = jnp.concatenate([wr[:, :1], wr[:, :T - 1]], axis=1)
    s_p1 = jnp.concatenate([wr[:, 1:], wr[:, T - 1:]], axis=1)
    s_p2 = jnp.concatenate([wr[:, 2:], wr[:, T - 1:], wr[:, T - 1:]], axis=1)
    wb = (s_m2 * c + s_m1 * c) + (wr * c + s_p1 * c) + s_p2 * c

    bu = lax.bitcast_convert_type(wb, jnp.uint32)
    key = jnp.where(wb < 0, ~bu, bu | jnp.uint32(0x80000000))
    rows = w.shape[0]
    ans = jnp.zeros((rows, 1), jnp.uint32)
    kf = jnp.float32(K_LOW)
    for bit in range(31, -1, -1):
        t = ans | jnp.uint32(1 << bit)
        cnt = jnp.sum(jnp.where(key < t, one, zero), axis=1, keepdims=True)
        ans = jnp.where(cnt <= kf, t, ans)
    v_low_u = jnp.where(ans >= jnp.uint32(0x80000000),
                        ans & jnp.uint32(0x7fffffff), ~ans)
    v_low = lax.bitcast_convert_type(v_low_u, jnp.float32)
    INF = jnp.float32(np.inf)
    c_le = jnp.sum(jnp.where(wb <= v_low, one, zero), axis=1, keepdims=True)
    gt_min = jnp.min(jnp.where(wb > v_low, wb, INF), axis=1, keepdims=True)
    v_high = jnp.where(c_le >= jnp.float32(K_LOW + 2), v_low, gt_min)
    tau = v_low * jnp.float32(LW) + v_high * jnp.float32(HW)

    m = jnp.where(wb >= tau, one, zero)

    and4 = m * _shift_l(m, 1) * _shift_l(m, 2) * _shift_l(m, 3)
    m2 = jnp.maximum(jnp.maximum(and4, _shift_r(and4, 1)),
                     jnp.maximum(_shift_r(and4, 2), _shift_r(and4, 3)))
    f1 = _shift_r(m2, 1) * _shift_l(m2, 1)
    f2 = _shift_r(m2, 1) * _shift_l(m2, 2)
    f3 = _shift_r(m2, 2) * _shift_l(m2, 1)
    m3 = jnp.maximum(jnp.maximum(m2, f1), jnp.maximum(f2, f3))

    start3 = m3 * (one - _shift_r(m3, 1))
    end3 = m3 * (one - _shift_l(m3, 1))
    iota = lax.broadcasted_iota(jnp.int32, (rows, T), 1).astype(jnp.float32)
    rid = _maxscan_hs(jnp.where(start3 > 0, iota, jnp.float32(-1.0)))
    seg = _segsum_hs(wb * m3, rid)
    pk = iota * jnp.float32(4096.0) + rid
    key_s = jnp.where(end3 > 0, seg, jnp.float32(NEG))

    E = jnp.exp(wb)
    BIGP = jnp.float32(T * 4096.0 * 2.0)
    any_valid = jnp.max(m3, axis=1, keepdims=True) > 0

    out_rows = []
    kcur = key_s
    for j in range(TOP_M):
        mx = jnp.max(kcur, axis=1, keepdims=True)
        pp = jnp.min(jnp.where(kcur == mx, pk, BIGP), axis=1, keepdims=True)
        valid = mx > jnp.float32(-1e29)
        e_j = jnp.floor(pp * jnp.float32(1.0 / 4096.0))
        s_j = pp - e_j * jnp.float32(4096.0)
        memb = jnp.where((iota >= s_j) & (iota <= e_j), one, zero)
        wsum = jnp.sum(memb * E, axis=1, keepdims=True)
        row = memb * E * (one / wsum)
        row = jnp.where(valid, row, zero)
        if j == 0:
            row = jnp.where(any_valid, row, jnp.float32(1.0 / T))
        out_rows.append(row.reshape(rows, 1, T))
        kcur = jnp.where(memb > 0, jnp.float32(NEG), kcur)
    return jnp.concatenate(out_rows, axis=1)


def _body(w_ref, vid_ref, out_ref, ww_s):
    i = pl.program_id(0)

    @pl.when(i == 0)
    def _():
        ww_s[...] = _weights_allbatch(w_ref[...]).astype(jnp.bfloat16)

    @pl.when(i > 0)
    def _():
        b = i - 1
        for r in range(VB):
            ww = ww_s[VB * b + r]
            vid = vid_ref[r].astype(jnp.bfloat16)
            out_ref[r] = jnp.dot(ww, vid, preferred_element_type=jnp.float32)


VB = 2


def kernel(w_bt, video):
    return pl.pallas_call(
        _body,
        grid=(B // VB + 1,),
        in_specs=[
            pl.BlockSpec((B, T), lambda i: (0, 0)),
            pl.BlockSpec((VB, T, D), lambda i: (jnp.maximum(i - 1, 0), 0, 0)),
        ],
        out_specs=pl.BlockSpec((VB, TOP_M, D), lambda i: (jnp.maximum(i - 1, 0), 0, 0)),
        out_shape=jax.ShapeDtypeStruct((B, TOP_M, D), jnp.float32),
        scratch_shapes=[pltpu.VMEM((B, TOP_M, T), jnp.bfloat16)],
        compiler_params=pltpu.CompilerParams(
            dimension_semantics=("arbitrary",),
            vmem_limit_bytes=100 * 1024 * 1024,
        ),
    )(w_bt, video)


# speedup vs baseline: 1.0763x; 1.0763x over previous
import numpy as np
import jax
import jax.numpy as jnp
from jax import lax
from jax.experimental import pallas as pl
from jax.experimental.pallas import tpu as pltpu

B, T, D = 32, 2048, 768
TOP_M = 8
NEG = -1e30
K_LOW = 1432
_POS = np.float32(0.7) * np.float32(2047.0)
HW = float(_POS - np.float32(np.floor(_POS)))
LW = float(np.float32(1.0) - np.float32(HW))


def _shift_r(x, k):
    r = x.shape[0]
    return jnp.concatenate([jnp.zeros((r, k), x.dtype), x[:, :T - k]], axis=1)


def _shift_l(x, k):
    r = x.shape[0]
    return jnp.concatenate([x[:, k:], jnp.zeros((r, k), x.dtype)], axis=1)


def _maxscan_hs(x):
    y = x
    k = 1
    while k < T:
        r = x.shape[0]
        sh = jnp.concatenate([jnp.full((r, k), -1.0, x.dtype), y[:, :T - k]], axis=1)
        y = jnp.maximum(y, sh)
        k *= 2
    return y


def _segsum_hs(x, rid):
    y = x
    k = 1
    while k < T:
        same = jnp.where(_shift_r(rid, k) == rid, jnp.float32(1.0), jnp.float32(0.0))
        y = y + same * _shift_r(y, k)
        k *= 2
    return y


def _weights_allbatch(w):
    one = jnp.float32(1.0)
    zero = jnp.float32(0.0)
    wr = w.astype(jnp.bfloat16).astype(jnp.float32)
    c = jnp.float32(0.2001953125)

    s_m2 = jnp.concatenate([wr[:, :1], wr[:, :1], wr[:, :T - 2]], axis=1)
    s_m1 = jnp.concatenate([wr[:, :1], wr[:, :T - 1]], axis=1)
    s_p1 = jnp.concatenate([wr[:, 1:], wr[:, T - 1:]], axis=1)
    s_p2 = jnp.concatenate([wr[:, 2:], wr[:, T - 1:], wr[:, T - 1:]], axis=1)
    wb = (s_m2 * c + s_m1 * c) + (wr * c + s_p1 * c) + s_p2 * c

    bu = lax.bitcast_convert_type(wb, jnp.uint32)
    key = jnp.where(wb < 0, ~bu, bu | jnp.uint32(0x80000000))
    rows = w.shape[0]
    ans = jnp.zeros((rows, 1), jnp.uint32)
    kf = jnp.float32(K_LOW)
    for bit in range(31, -1, -1):
        t = ans | jnp.uint32(1 << bit)
        cnt = jnp.sum(jnp.where(key < t, one, zero), axis=1, keepdims=True)
        ans = jnp.where(cnt <= kf, t, ans)
    v_low_u = jnp.where(ans >= jnp.uint32(0x80000000),
                        ans & jnp.uint32(0x7fffffff), ~ans)
    v_low = lax.bitcast_convert_type(v_low_u, jnp.float32)
    INF = jnp.float32(np.inf)
    c_le = jnp.sum(jnp.where(wb <= v_low, one, zero), axis=1, keepdims=True)
    gt_min = jnp.min(jnp.where(wb > v_low, wb, INF), axis=1, keepdims=True)
    v_high = jnp.where(c_le >= jnp.float32(K_LOW + 2), v_low, gt_min)
    tau = v_low * jnp.float32(LW) + v_high * jnp.float32(HW)

    m = jnp.where(wb >= tau, one, zero)

    and4 = m * _shift_l(m, 1) * _shift_l(m, 2) * _shift_l(m, 3)
    m2 = jnp.maximum(jnp.maximum(and4, _shift_r(and4, 1)),
                     jnp.maximum(_shift_r(and4, 2), _shift_r(and4, 3)))
    f1 = _shift_r(m2, 1) * _shift_l(m2, 1)
    f2 = _shift_r(m2, 1) * _shift_l(m2, 2)
    f3 = _shift_r(m2, 2) * _shift_l(m2, 1)
    m3 = jnp.maximum(jnp.maximum(m2, f1), jnp.maximum(f2, f3))

    start3 = m3 * (one - _shift_r(m3, 1))
    end3 = m3 * (one - _shift_l(m3, 1))
    iota = lax.broadcasted_iota(jnp.int32, (rows, T), 1).astype(jnp.float32)
    rid = _maxscan_hs(jnp.where(start3 > 0, iota, jnp.float32(-1.0)))
    seg = _segsum_hs(wb * m3, rid)
    pk = iota * jnp.float32(4096.0) + rid
    key_s = jnp.where(end3 > 0, seg, jnp.float32(NEG))

    E = jnp.exp(wb)
    BIGP = jnp.float32(T * 4096.0 * 2.0)
    any_valid = jnp.max(m3, axis=1, keepdims=True) > 0

    out_rows = []
    kcur = key_s
    for j in range(TOP_M):
        mx = jnp.max(kcur, axis=1, keepdims=True)
        pp = jnp.min(jnp.where(kcur == mx, pk, BIGP), axis=1, keepdims=True)
        valid = mx > jnp.float32(-1e29)
        e_j = jnp.floor(pp * jnp.float32(1.0 / 4096.0))
        s_j = pp - e_j * jnp.float32(4096.0)
        memb = jnp.where((iota >= s_j) & (iota <= e_j), one, zero)
        wsum = jnp.sum(memb * E, axis=1, keepdims=True)
        row = memb * E * (one / wsum)
        row = jnp.where(valid, row, zero)
        if j == 0:
            row = jnp.where(any_valid, row, jnp.float32(1.0 / T))
        out_rows.append(row.reshape(rows, 1, T))
        kcur = jnp.where(memb > 0, jnp.float32(NEG), kcur)
    return jnp.concatenate(out_rows, axis=1)


NBUF = 6


def _body(w_ref, vid_hbm, out_ref, ww_s, vbuf, sems):
    i = pl.program_id(0)

    @pl.when(i == 0)
    def _():
        for k in range(NBUF):
            pltpu.make_async_copy(vid_hbm.at[k], vbuf.at[k], sems.at[k]).start()
        ww_s[...] = _weights_allbatch(w_ref[...]).astype(jnp.bfloat16)

    @pl.when(i > 0)
    def _():
        b = i - 1
        slot = lax.rem(b, NBUF)
        pltpu.make_async_copy(vbuf.at[slot], vbuf.at[slot], sems.at[slot]).wait()
        ww = ww_s[b]
        vid = vbuf[slot].astype(jnp.bfloat16)
        out_ref[0] = jnp.dot(ww, vid, preferred_element_type=jnp.float32)
        nxt = b + NBUF

        @pl.when(nxt < B)
        def _():
            pltpu.make_async_copy(vid_hbm.at[nxt], vbuf.at[slot],
                                  sems.at[slot]).start()


def kernel(w_bt, video):
    return pl.pallas_call(
        _body,
        grid=(B + 1,),
        in_specs=[
            pl.BlockSpec((B, T), lambda i: (0, 0)),
            pl.BlockSpec(memory_space=pl.ANY),
        ],
        out_specs=pl.BlockSpec((1, TOP_M, D), lambda i: (jnp.maximum(i - 1, 0), 0, 0)),
        out_shape=jax.ShapeDtypeStruct((B, TOP_M, D), jnp.float32),
        scratch_shapes=[pltpu.VMEM((B, TOP_M, T), jnp.bfloat16),
                        pltpu.VMEM((NBUF, T, D), jnp.float32),
                        pltpu.SemaphoreType.DMA((NBUF,))],
        compiler_params=pltpu.CompilerParams(
            dimension_semantics=("arbitrary",),
            vmem_limit_bytes=100 * 1024 * 1024,
        ),
    )(w_bt, video)


# speedup vs baseline: 1.0860x; 1.0090x over previous
import numpy as np
import jax
import jax.numpy as jnp
from jax import lax
from jax.experimental import pallas as pl
from jax.experimental.pallas import tpu as pltpu

B, T, D = 32, 2048, 768
TOP_M = 8
NEG = -1e30
K_LOW = 1432
_POS = np.float32(0.7) * np.float32(2047.0)
HW = float(_POS - np.float32(np.floor(_POS)))
LW = float(np.float32(1.0) - np.float32(HW))


def _shift_r(x, k):
    r = x.shape[0]
    return jnp.concatenate([jnp.zeros((r, k), x.dtype), x[:, :T - k]], axis=1)


def _shift_l(x, k):
    r = x.shape[0]
    return jnp.concatenate([x[:, k:], jnp.zeros((r, k), x.dtype)], axis=1)


def _maxscan_hs(x):
    y = x
    k = 1
    while k < T:
        r = x.shape[0]
        sh = jnp.concatenate([jnp.full((r, k), -1.0, x.dtype), y[:, :T - k]], axis=1)
        y = jnp.maximum(y, sh)
        k *= 2
    return y


def _segsum_hs(x, rid):
    y = x
    k = 1
    while k < T:
        same = jnp.where(_shift_r(rid, k) == rid, jnp.float32(1.0), jnp.float32(0.0))
        y = y + same * _shift_r(y, k)
        k *= 2
    return y


def _weights_allbatch(w):
    one = jnp.float32(1.0)
    zero = jnp.float32(0.0)
    wr = w.astype(jnp.bfloat16).astype(jnp.float32)
    c = jnp.float32(0.2001953125)

    s_m2 = jnp.concatenate([wr[:, :1], wr[:, :1], wr[:, :T - 2]], axis=1)
    s_m1 = jnp.concatenate([wr[:, :1], wr[:, :T - 1]], axis=1)
    s_p1 = jnp.concatenate([wr[:, 1:], wr[:, T - 1:]], axis=1)
    s_p2 = jnp.concatenate([wr[:, 2:], wr[:, T - 1:], wr[:, T - 1:]], axis=1)
    wb = (s_m2 * c + s_m1 * c) + (wr * c + s_p1 * c) + s_p2 * c

    bu = lax.bitcast_convert_type(wb, jnp.uint32)
    key = jnp.where(wb < 0, ~bu, bu | jnp.uint32(0x80000000))
    rows = w.shape[0]
    ans = jnp.zeros((rows, 1), jnp.uint32)
    kf = jnp.float32(K_LOW)
    for bit in range(31, -1, -1):
        t = ans | jnp.uint32(1 << bit)
        cnt = jnp.sum(jnp.where(key < t, one, zero), axis=1, keepdims=True)
        ans = jnp.where(cnt <= kf, t, ans)
    v_low_u = jnp.where(ans >= jnp.uint32(0x80000000),
                        ans & jnp.uint32(0x7fffffff), ~ans)
    v_low = lax.bitcast_convert_type(v_low_u, jnp.float32)
    INF = jnp.float32(np.inf)
    c_le = jnp.sum(jnp.where(wb <= v_low, one, zero), axis=1, keepdims=True)
    gt_min = jnp.min(jnp.where(wb > v_low, wb, INF), axis=1, keepdims=True)
    v_high = jnp.where(c_le >= jnp.float32(K_LOW + 2), v_low, gt_min)
    tau = v_low * jnp.float32(LW) + v_high * jnp.float32(HW)

    m = jnp.where(wb >= tau, one, zero)

    and4 = m * _shift_l(m, 1) * _shift_l(m, 2) * _shift_l(m, 3)
    m2 = jnp.maximum(jnp.maximum(and4, _shift_r(and4, 1)),
                     jnp.maximum(_shift_r(and4, 2), _shift_r(and4, 3)))
    f1 = _shift_r(m2, 1) * _shift_l(m2, 1)
    f2 = _shift_r(m2, 1) * _shift_l(m2, 2)
    f3 = _shift_r(m2, 2) * _shift_l(m2, 1)
    m3 = jnp.maximum(jnp.maximum(m2, f1), jnp.maximum(f2, f3))

    start3 = m3 * (one - _shift_r(m3, 1))
    end3 = m3 * (one - _shift_l(m3, 1))
    iota = lax.broadcasted_iota(jnp.int32, (rows, T), 1).astype(jnp.float32)
    rid = _maxscan_hs(jnp.where(start3 > 0, iota, jnp.float32(-1.0)))
    seg = _segsum_hs(wb * m3, rid)
    pk = iota * jnp.float32(4096.0) + rid
    key_s = jnp.where(end3 > 0, seg, jnp.float32(NEG))

    E = jnp.exp(wb)
    BIGP = jnp.float32(T * 4096.0 * 2.0)
    any_valid = jnp.max(m3, axis=1, keepdims=True) > 0

    out_rows = []
    kcur = key_s
    for j in range(TOP_M):
        mx = jnp.max(kcur, axis=1, keepdims=True)
        pp = jnp.min(jnp.where(kcur == mx, pk, BIGP), axis=1, keepdims=True)
        valid = mx > jnp.float32(-1e29)
        e_j = jnp.floor(pp * jnp.float32(1.0 / 4096.0))
        s_j = pp - e_j * jnp.float32(4096.0)
        memb = jnp.where((iota >= s_j) & (iota <= e_j), one, zero)
        wsum = jnp.sum(memb * E, axis=1, keepdims=True)
        row = memb * E * (one / wsum)
        row = jnp.where(valid, row, zero)
        if j == 0:
            row = jnp.where(any_valid, row, jnp.float32(1.0 / T))
        out_rows.append(row.reshape(rows, 1, T))
        kcur = jnp.where(memb > 0, jnp.float32(NEG), kcur)
    return jnp.concatenate(out_rows, axis=1)


NBUF = 8


def _body(w_ref, vid_hbm, out_ref, ww_s, vbuf, sems):
    i = pl.program_id(0)

    @pl.when(i == 0)
    def _():
        for k in range(NBUF):
            pltpu.make_async_copy(vid_hbm.at[k], vbuf.at[k], sems.at[k]).start()
        ww_s[...] = _weights_allbatch(w_ref[...]).astype(jnp.bfloat16)

    @pl.when(i > 0)
    def _():
        b = i - 1
        slot = lax.rem(b, NBUF)
        pltpu.make_async_copy(vbuf.at[slot], vbuf.at[slot], sems.at[slot]).wait()
        ww = ww_s[b]
        vid = vbuf[slot].astype(jnp.bfloat16)
        out_ref[0] = jnp.dot(ww, vid, preferred_element_type=jnp.float32)
        nxt = b + NBUF

        @pl.when(nxt < B)
        def _():
            pltpu.make_async_copy(vid_hbm.at[nxt], vbuf.at[slot],
                                  sems.at[slot]).start()


def kernel(w_bt, video):
    return pl.pallas_call(
        _body,
        grid=(B + 1,),
        in_specs=[
            pl.BlockSpec((B, T), lambda i: (0, 0)),
            pl.BlockSpec(memory_space=pl.ANY),
        ],
        out_specs=pl.BlockSpec((1, TOP_M, D), lambda i: (jnp.maximum(i - 1, 0), 0, 0)),
        out_shape=jax.ShapeDtypeStruct((B, TOP_M, D), jnp.float32),
        scratch_shapes=[pltpu.VMEM((B, TOP_M, T), jnp.bfloat16),
                        pltpu.VMEM((NBUF, T, D), jnp.float32),
                        pltpu.SemaphoreType.DMA((NBUF,))],
        compiler_params=pltpu.CompilerParams(
            dimension_semantics=("arbitrary",),
            vmem_limit_bytes=100 * 1024 * 1024,
        ),
    )(w_bt, video)
